# baseline (speedup 1.0000x reference)
import functools

import jax
import jax.numpy as jnp
from jax import lax
from jax.experimental import pallas as pl
from jax.experimental.pallas import tpu as pltpu
from jax.experimental.pallas import tpu_sc as plsc

C = 4
K = 1024
HD = 512
TOK_TILE = 512
N_SLABS = 2


def _ids_kernel(x_ref, cb_ref, ids_ref, cbsq_ref):
    @pl.when(pl.program_id(0) == 0)
    def _():
        for c in range(C):
            cb = cb_ref[c]
            cbsq_ref[c, :] = jnp.sum(cb * cb, axis=1)

    xb = x_ref[...]
    rows = []
    for c in range(C):
        xc = xb[:, c * HD:(c + 1) * HD]
        cb = cb_ref[c]
        comp_sq = jnp.sum(xc * xc, axis=1, keepdims=True)
        cb_sq = cbsq_ref[c, :][None, :]
        cross = lax.dot_general(
            xc, cb, (((1,), (1,)), ((), ())),
            preferred_element_type=jnp.float32)
        d2 = jnp.maximum((comp_sq + cb_sq) - 2.0 * cross, 0.0)
        dist = jnp.sqrt(d2)
        m = jnp.min(dist, axis=1, keepdims=True)
        iota = lax.broadcasted_iota(jnp.int32, dist.shape, 1)
        idx = jnp.min(jnp.where(dist == m, iota, K), axis=1)
        rows.append((idx + c * K)[None, :])
    ids_ref[...] = jnp.concatenate(rows, axis=0)


def _compute_ids(x2d, codebook, slab, slab_tok):
    tiles_per_slab = slab_tok // TOK_TILE
    return pl.pallas_call(
        _ids_kernel,
        grid=(tiles_per_slab,),
        in_specs=[
            pl.BlockSpec((TOK_TILE, C * HD),
                         lambda i: (slab * tiles_per_slab + i, 0)),
            pl.BlockSpec((C, K, HD), lambda i: (0, 0, 0)),
        ],
        out_specs=pl.BlockSpec((C, TOK_TILE), lambda i: (0, i)),
        out_shape=jax.ShapeDtypeStruct((C, slab_tok), jnp.int32),
        scratch_shapes=[pltpu.VMEM((C, K), jnp.float32)],
    )(x2d, codebook)


def _make_gather(slab_tok, t_offset):
    info = plsc.get_sparse_core_info()
    nc, ns = info.num_cores, info.num_subcores
    nw = nc * ns
    tb = 32
    per_w = slab_tok // tb // nw
    tok_per_w = slab_tok // nw
    mesh = plsc.VectorSubcoreMesh(core_axis_name="c", subcore_axis_name="s")

    @functools.partial(
        pl.kernel, mesh=mesh,
        out_type=(),
    scratch_types=[
            pltpu.VMEM((C, slab_tok), jnp.int32),
            pltpu.VMEM((tb, C * HD), jnp.float32),
            pltpu.SemaphoreType.DMA,
        ],
    )
    def gather_k(table_hbm, ids_hbm, out_hbm, stage_v, rows_v, sem):
        wid = lax.axis_index("s") * nc + lax.axis_index("c")
        t_base = wid * tok_per_w
        pltpu.sync_copy(ids_hbm, stage_v)
        for j in range(per_w):
            cps = [pltpu.async_copy(
                table_hbm.at[stage_v.at[c, pl.ds(t_base + j * tb, tb)]],
                rows_v.at[:, pl.ds(c * HD, HD)], sem) for c in range(C)]
            for cp in cps:
                cp.wait()
            pltpu.sync_copy(
                rows_v, out_hbm.at[pl.ds(t_offset + t_base + j * tb, tb)])

    return gather_k


def kernel(x, codebook):
    B, S, D = x.shape
    n_tok = B * S
    slab_tok = n_tok // N_SLABS
    x2d = x.reshape(n_tok, D)
    table = codebook.reshape(C * K, HD)
    out_ref = jax.new_ref(jnp.zeros((n_tok, D), jnp.float32))
    for slab in range(N_SLABS):
        ids = _compute_ids(x2d, codebook, slab, slab_tok)
        _make_gather(slab_tok, slab * slab_tok)(table, ids, out_ref)
    return out_ref[...].reshape(B, S, D)


# speedup vs baseline: 1.1037x; 1.1037x over previous
import functools

import jax
import jax.numpy as jnp
from jax import lax
from jax.experimental import pallas as pl
from jax.experimental.pallas import tpu as pltpu
from jax.experimental.pallas import tpu_sc as plsc

C = 4
K = 1024
HD = 512
TOK_TILE = 512
N_SLABS = 2


def _ids_kernel(x_ref, cb_ref, ids_ref, cbsq_ref):
    @pl.when(pl.program_id(0) == 0)
    def _():
        for c in range(C):
            cb = cb_ref[c]
            cbsq_ref[c, :] = jnp.sum(cb * cb, axis=1)

    xb = x_ref[...]
    rows = []
    for c in range(C):
        xc = xb[:, c * HD:(c + 1) * HD]
        cb = cb_ref[c]
        comp_sq = jnp.sum(xc * xc, axis=1, keepdims=True)
        cb_sq = cbsq_ref[c, :][None, :]
        cross = lax.dot_general(
            xc, cb, (((1,), (1,)), ((), ())),
            preferred_element_type=jnp.float32)
        d2 = jnp.maximum((comp_sq + cb_sq) - 2.0 * cross, 0.0)
        dist = jnp.sqrt(d2)
        m = jnp.min(dist, axis=1, keepdims=True)
        iota = lax.broadcasted_iota(jnp.int32, dist.shape, 1)
        idx = jnp.min(jnp.where(dist == m, iota, K), axis=1)
        rows.append((idx + c * K)[None, :])
    ids_ref[...] = jnp.concatenate(rows, axis=0)


def _compute_ids(x2d, codebook, slab, slab_tok):
    tiles_per_slab = slab_tok // TOK_TILE
    return pl.pallas_call(
        _ids_kernel,
        grid=(tiles_per_slab,),
        in_specs=[
            pl.BlockSpec((TOK_TILE, C * HD),
                         lambda i: (slab * tiles_per_slab + i, 0)),
            pl.BlockSpec((C, K, HD), lambda i: (0, 0, 0)),
        ],
        out_specs=pl.BlockSpec((C, TOK_TILE), lambda i: (0, i)),
        out_shape=jax.ShapeDtypeStruct((C, slab_tok), jnp.int32),
        scratch_shapes=[pltpu.VMEM((C, K), jnp.float32)],
    )(x2d, codebook)


def _make_gather(slab_tok, t_offset):
    info = plsc.get_sparse_core_info()
    nc, ns = info.num_cores, info.num_subcores
    nw = nc * ns
    tb = 32
    per_w = slab_tok // tb // nw
    tok_per_w = slab_tok // nw
    mesh = plsc.VectorSubcoreMesh(core_axis_name="c", subcore_axis_name="s")

    @functools.partial(
        pl.kernel, mesh=mesh,
        out_type=(),
    scratch_types=[
            pltpu.VMEM((C, slab_tok), jnp.int32),
            pltpu.VMEM((tb, C * HD), jnp.float32),
            pltpu.SemaphoreType.DMA,
        ],
    )
    def gather_k(table_hbm, ids_hbm, out_hbm, stage_v, rows_v, sem):
        wid = lax.axis_index("s") * nc + lax.axis_index("c")
        t_base = wid * tok_per_w
        pltpu.sync_copy(ids_hbm, stage_v)
        for j in range(per_w):
            cps = [pltpu.async_copy(
                table_hbm.at[stage_v.at[c, pl.ds(t_base + j * tb, tb)]],
                rows_v.at[:, pl.ds(c * HD, HD)], sem) for c in range(C)]
            for cp in cps:
                cp.wait()
            pltpu.sync_copy(
                rows_v, out_hbm.at[pl.ds(t_offset + t_base + j * tb, tb)])

    return gather_k


def kernel(x, codebook):
    B, S, D = x.shape
    n_tok = B * S
    slab_tok = n_tok // N_SLABS
    x2d = x.reshape(n_tok, D)
    table = codebook.reshape(C * K, HD)
    out_ref = jax.new_ref(lax.empty((n_tok, D), jnp.float32))
    for slab in range(N_SLABS):
        ids = _compute_ids(x2d, codebook, slab, slab_tok)
        _make_gather(slab_tok, slab * slab_tok)(table, ids, out_ref)
    return out_ref[...].reshape(B, S, D)


# speedup vs baseline: 1.1387x; 1.0317x over previous
import functools

import jax
import jax.numpy as jnp
from jax import lax
from jax.experimental import pallas as pl
from jax.experimental.pallas import tpu as pltpu
from jax.experimental.pallas import tpu_sc as plsc

C = 4
K = 1024
HD = 512
TOK_TILE = 512
N_SLABS = 2


def _ids_kernel(x_ref, cb_ref, ids_ref, cbsq_ref):
    @pl.when(pl.program_id(0) == 0)
    def _():
        for c in range(C):
            cb = cb_ref[c]
            cbsq_ref[c, :] = jnp.sum(cb * cb, axis=1)

    xb = x_ref[...]
    rows = []
    for c in range(C):
        xc = xb[:, c * HD:(c + 1) * HD]
        cb = cb_ref[c]
        comp_sq = jnp.sum(xc * xc, axis=1, keepdims=True)
        cb_sq = cbsq_ref[c, :][None, :]
        cross = lax.dot_general(
            xc, cb, (((1,), (1,)), ((), ())),
            preferred_element_type=jnp.float32)
        d2 = jnp.maximum((comp_sq + cb_sq) - 2.0 * cross, 0.0)
        dist = jnp.sqrt(d2)
        idx = jnp.argmin(dist, axis=1).astype(jnp.int32)
        rows.append((idx + c * K)[None, :])
    ids_ref[...] = jnp.concatenate(rows, axis=0)


def _compute_ids(x2d, codebook, slab, slab_tok):
    tiles_per_slab = slab_tok // TOK_TILE
    return pl.pallas_call(
        _ids_kernel,
        grid=(tiles_per_slab,),
        in_specs=[
            pl.BlockSpec((TOK_TILE, C * HD),
                         lambda i: (slab * tiles_per_slab + i, 0)),
            pl.BlockSpec((C, K, HD), lambda i: (0, 0, 0)),
        ],
        out_specs=pl.BlockSpec((C, TOK_TILE), lambda i: (0, i)),
        out_shape=jax.ShapeDtypeStruct((C, slab_tok), jnp.int32),
        scratch_shapes=[pltpu.VMEM((C, K), jnp.float32)],
    )(x2d, codebook)


def _make_gather(slab_tok, t_offset):
    info = plsc.get_sparse_core_info()
    nc, ns = info.num_cores, info.num_subcores
    nw = nc * ns
    tb = 32
    per_w = slab_tok // tb // nw
    tok_per_w = slab_tok // nw
    mesh = plsc.VectorSubcoreMesh(core_axis_name="c", subcore_axis_name="s")

    @functools.partial(
        pl.kernel, mesh=mesh,
        out_type=(),
    scratch_types=[
            pltpu.VMEM((C, slab_tok), jnp.int32),
            pltpu.VMEM((tb, C * HD), jnp.float32),
            pltpu.SemaphoreType.DMA,
        ],
    )
    def gather_k(table_hbm, ids_hbm, out_hbm, stage_v, rows_v, sem):
        wid = lax.axis_index("s") * nc + lax.axis_index("c")
        t_base = wid * tok_per_w
        pltpu.sync_copy(ids_hbm, stage_v)
        for j in range(per_w):
            cps = [pltpu.async_copy(
                table_hbm.at[stage_v.at[c, pl.ds(t_base + j * tb, tb)]],
                rows_v.at[:, pl.ds(c * HD, HD)], sem) for c in range(C)]
            for cp in cps:
                cp.wait()
            pltpu.sync_copy(
                rows_v, out_hbm.at[pl.ds(t_offset + t_base + j * tb, tb)])

    return gather_k


def kernel(x, codebook):
    B, S, D = x.shape
    n_tok = B * S
    slab_tok = n_tok // N_SLABS
    x2d = x.reshape(n_tok, D)
    table = codebook.reshape(C * K, HD)
    out_ref = jax.new_ref(lax.empty((n_tok, D), jnp.float32))
    for slab in range(N_SLABS):
        ids = _compute_ids(x2d, codebook, slab, slab_tok)
        _make_gather(slab_tok, slab * slab_tok)(table, ids, out_ref)
    return out_ref[...].reshape(B, S, D)


# speedup vs baseline: 1.1506x; 1.0104x over previous
import functools

import jax
import jax.numpy as jnp
from jax import lax
from jax.experimental import pallas as pl
from jax.experimental.pallas import tpu as pltpu
from jax.experimental.pallas import tpu_sc as plsc

C = 4
K = 1024
HD = 512
TOK_TILE = 512
N_SLABS = 2


def _ids_kernel(x_ref, cb_ref, ids_ref, cbsq_ref):
    @pl.when(pl.program_id(0) == 0)
    def _():
        for c in range(C):
            cb = cb_ref[c]
            cbsq_ref[c, :] = jnp.sum(cb * cb, axis=1)

    xb = x_ref[...]
    rows = []
    for c in range(C):
        xc = xb[:, c * HD:(c + 1) * HD]
        cb = cb_ref[c]
        comp_sq = jnp.sum(xc * xc, axis=1, keepdims=True)
        cb_sq = cbsq_ref[c, :][None, :]
        cross = lax.dot_general(
            xc, cb, (((1,), (1,)), ((), ())),
            preferred_element_type=jnp.float32)
        d2 = jnp.maximum((comp_sq + cb_sq) - 2.0 * cross, 0.0)
        dist = jnp.sqrt(d2)
        idx = jnp.argmin(dist, axis=1).astype(jnp.int32)
        rows.append((idx + c * K)[None, :])
    ids_ref[...] = jnp.concatenate(rows, axis=0)


def _compute_ids(x2d, codebook, slab, slab_tok):
    tiles_per_slab = slab_tok // TOK_TILE
    return pl.pallas_call(
        _ids_kernel,
        grid=(tiles_per_slab,),
        in_specs=[
            pl.BlockSpec((TOK_TILE, C * HD),
                         lambda i: (slab * tiles_per_slab + i, 0)),
            pl.BlockSpec((C, K, HD), lambda i: (0, 0, 0)),
        ],
        out_specs=pl.BlockSpec((C, TOK_TILE), lambda i: (0, i)),
        out_shape=jax.ShapeDtypeStruct((C, slab_tok), jnp.int32),
        scratch_shapes=[pltpu.VMEM((C, K), jnp.float32)],
    )(x2d, codebook)


def _make_gather(slab_tok, t_offset):
    info = plsc.get_sparse_core_info()
    nc, ns = info.num_cores, info.num_subcores
    nw = nc * ns
    tb = 16
    per_w = slab_tok // tb // nw
    tok_per_w = slab_tok // nw
    mesh = plsc.VectorSubcoreMesh(core_axis_name="c", subcore_axis_name="s")

    @functools.partial(
        pl.kernel, mesh=mesh,
        out_type=(),
        scratch_types=[
            pltpu.VMEM((C, slab_tok), jnp.int32),
            pltpu.VMEM((2, tb, C * HD), jnp.float32),
            pltpu.SemaphoreType.DMA,
            pltpu.SemaphoreType.DMA,
            pltpu.SemaphoreType.DMA,
            pltpu.SemaphoreType.DMA,
        ],
    )
    def gather_k(table_hbm, ids_hbm, out_hbm, stage_v, rows_v, g0, g1, w0, w1):
        wid = lax.axis_index("s") * nc + lax.axis_index("c")
        t_base = wid * tok_per_w
        pltpu.sync_copy(ids_hbm, stage_v)
        gsem, wsem = (g0, g1), (w0, w1)
        writes = [None, None]
        for j in range(per_w):
            p = j % 2
            if writes[p] is not None:
                writes[p].wait()
            cps = [pltpu.async_copy(
                table_hbm.at[stage_v.at[c, pl.ds(t_base + j * tb, tb)]],
                rows_v.at[p, :, pl.ds(c * HD, HD)], gsem[p]) for c in range(C)]
            for cp in cps:
                cp.wait()
            writes[p] = pltpu.async_copy(
                rows_v.at[p], out_hbm.at[pl.ds(t_offset + t_base + j * tb, tb)],
                wsem[p])
        for w in writes:
            if w is not None:
                w.wait()

    return gather_k


def kernel(x, codebook):
    B, S, D = x.shape
    n_tok = B * S
    slab_tok = n_tok // N_SLABS
    x2d = x.reshape(n_tok, D)
    table = codebook.reshape(C * K, HD)
    out_ref = jax.new_ref(lax.empty((n_tok, D), jnp.float32))
    for slab in range(N_SLABS):
        ids = _compute_ids(x2d, codebook, slab, slab_tok)
        _make_gather(slab_tok, slab * slab_tok)(table, ids, out_ref)
    return out_ref[...].reshape(B, S, D)
